# Initial kernel scaffold; baseline (speedup 1.0000x reference)
import jax
import jax.numpy as jnp
from jax.experimental import pallas as pl


def kernel(feat, edge_index, edge_feat, params):
    raise NotImplementedError("write your pallas kernel here")


# speedup vs baseline: 6.8691x; 6.8691x over previous
import functools

import jax
import jax.numpy as jnp
from jax import lax
from jax.experimental import pallas as pl
from jax.experimental.pallas import tpu as pltpu
from jax.experimental.pallas import tpu_sc as plsc

N = 50000
DEG = 5
D = 128
DE = 16
H = 8
DIM = 64
E = N * DEG
HEAD = D // H
SCALE = HEAD ** -0.5

_C = 80
_NCHUNK = E // _C
_NW = 32
_CPW = -(-_NCHUNK // _NW)
_EP = _NW * _CPW * _C

_TC_BN = 1000


def _sc_gather(table, idx3):
    mesh = plsc.VectorSubcoreMesh(core_axis_name="c", subcore_axis_name="s")

    @functools.partial(
        pl.kernel,
        mesh=mesh,
        out_type=jax.ShapeDtypeStruct((E, D), jnp.float32),
        scratch_types=[
            pltpu.VMEM((_CPW, _C), jnp.int32),
            pltpu.VMEM((_C, D), jnp.float32),
            pltpu.SemaphoreType.DMA,
        ],
    )
    def k(idx_hbm, table_hbm, out_hbm, idx_v, rows_v, sem):
        wid = lax.axis_index("s") * 2 + lax.axis_index("c")
        pltpu.sync_copy(idx_hbm.at[wid], idx_v)

        def body(i, carry):
            chunk = wid * _CPW + i

            @pl.when(chunk < _NCHUNK)
            def _():
                pltpu.async_copy(table_hbm.at[idx_v.at[i]], rows_v, sem).wait()
                pltpu.sync_copy(rows_v, out_hbm.at[pl.ds(chunk * _C, _C)])

            return carry

        lax.fori_loop(0, _CPW, body, 0)

    return k(idx3, table)


def _ln(x, g, b):
    m = jnp.mean(x, axis=-1, keepdims=True)
    xc = x - m
    v = jnp.mean(xc * xc, axis=-1, keepdims=True)
    return xc / jnp.sqrt(v + 1e-5) * g + b


def _dot(a, b):
    return jnp.dot(a, b, preferred_element_type=jnp.float32)


def _attn_layer(x, sa, gs, v1s, negs, S, B8):
    (Wq, Wk, Wv, Wo, bo, Wl1, bl1, Wl2, bl2, g1, be1, g2, be2) = sa
    q1 = _dot(x, Wq[...])
    dots = []
    for j in range(DEG):
        k1 = _dot(gs[j], Wk[...])
        d = _dot(q1 * k1, S) * SCALE
        dots.append(jnp.where(negs[j], -jnp.inf, d))
    m = dots[0]
    for j in range(1, DEG):
        m = jnp.maximum(m, dots[j])
    es = [jnp.exp(d - m) for d in dots]
    tot = es[0]
    for j in range(1, DEG):
        tot = tot + es[j]
    attn = None
    for j in range(DEG):
        w = _dot(es[j] / tot, B8)
        contrib = w * _dot(v1s[j], Wv[...])
        attn = contrib if attn is None else attn + contrib
    o = _dot(attn, Wo[...]) + bo[...]
    x = x + o
    x = _ln(x, g1[...], be1[...])
    ff = _dot(jnp.maximum(_dot(x, Wl1[...]) + bl1[...], 0.0), Wl2[...]) + bl2[...]
    x = x + ff
    return _ln(x, g2[...], be2[...])


def _tc_body(*refs):
    feat_ref, gath_ref, ef_ref = refs[0], refs[1], refs[2]
    w = refs[3:-1]
    out_ref = refs[-1]
    W1a, W1b, b1 = w[0], w[1], w[2]
    sa1 = w[3:16]
    sa2 = w[16:29]
    W3, b3 = w[29], w[30]

    rS = lax.broadcasted_iota(jnp.int32, (D, H), 0) // HEAD
    cS = lax.broadcasted_iota(jnp.int32, (D, H), 1)
    S = (rS == cS).astype(jnp.float32)
    rB = lax.broadcasted_iota(jnp.int32, (H, D), 0)
    cB = lax.broadcasted_iota(jnp.int32, (H, D), 1) // HEAD
    B8 = (rB == cB).astype(jnp.float32)

    x0 = feat_ref[...]
    gs, v1s, negs = [], [], []
    for j in range(DEG):
        g = gath_ref[j]
        gs.append(g)
        negs.append(jnp.sum(g, axis=1, keepdims=True) == 0.0)
        v1s.append(_dot(g, W1a[...]) + _dot(ef_ref[j], W1b[...]) + b1[...])

    x = _attn_layer(x0, sa1, gs, v1s, negs, S, B8)
    x = _attn_layer(x, sa2, gs, v1s, negs, S, B8)
    out_ref[...] = jnp.tanh(_dot(x, W3[...]) + b3[...])


def _full_spec(shape):
    nd = len(shape)
    return pl.BlockSpec(shape, lambda i, _n=nd: (0,) * _n)


def _sa_flat(p):
    return [
        p['Wq'], p['Wk'], p['Wv'], p['Wo'], p['bo'].reshape(1, D),
        p['Wl1'], p['bl1'].reshape(1, 2 * D), p['Wl2'], p['bl2'].reshape(1, D),
        p['g1'].reshape(1, D), p['be1'].reshape(1, D),
        p['g2'].reshape(1, D), p['be2'].reshape(1, D),
    ]


def _tc_forward(feat, gath3, ef_t, params):
    weights = (
        [params['W1'][:D], params['W1'][D:], params['b1'].reshape(1, D)]
        + _sa_flat(params['sa1'])
        + _sa_flat(params['sa2'])
        + [params['W3'], params['b3'].reshape(1, DIM)]
    )
    bn = _TC_BN
    in_specs = [
        pl.BlockSpec((bn, D), lambda i: (i, 0)),
        pl.BlockSpec((DEG, bn, D), lambda i: (0, i, 0)),
        pl.BlockSpec((DEG, bn, DE), lambda i: (0, i, 0)),
    ] + [_full_spec(x.shape) for x in weights]
    return pl.pallas_call(
        _tc_body,
        grid=(N // bn,),
        in_specs=in_specs,
        out_specs=pl.BlockSpec((bn, DIM), lambda i: (i, 0)),
        out_shape=jax.ShapeDtypeStruct((N, DIM), jnp.float32),
    )(feat, gath3, ef_t, *weights)


def kernel(feat, edge_index, edge_feat, params):
    src = edge_index[0]
    perm = src.reshape(N, DEG).T.reshape(-1)
    perm = jnp.concatenate([perm, jnp.zeros((_EP - E,), jnp.int32)])
    idx3 = perm.reshape(_NW, _CPW, _C)

    gath = _sc_gather(feat, idx3)
    gath3 = gath.reshape(DEG, N, D)
    ef_t = edge_feat.reshape(N, DEG, DE).transpose(1, 0, 2)

    return _tc_forward(feat, gath3, ef_t, params)
